# Initial kernel scaffold; baseline (speedup 1.0000x reference)
import jax
import jax.numpy as jnp
from jax.experimental import pallas as pl


def kernel(x, edge_idx, W1, b1, W2, b2):
    raise NotImplementedError("write your pallas kernel here")


# speedup vs baseline: 19.6757x; 19.6757x over previous
import functools

import jax
import jax.numpy as jnp
from jax import lax
from jax.experimental import pallas as pl
from jax.experimental.pallas import tpu as pltpu
from jax.experimental.pallas import tpu_sc as plsc

N = 10000
E = 320000
D = 128
H = 64
C = 16

NC = 2
NS = 16
NW = NC * NS
G = 128
GT = 80
EPAD = NW * GT * G
NPAD = 10240
RPT = NPAD // NS
DW = 16


def _zero_rows(ref, nrows, width):
  @pl.loop(0, nrows)
  def _(r):
    for c in range(width // 16):
      ref[r, pl.ds(c * 16, 16)] = jnp.zeros((16,), jnp.float32)


def _wid():
  return lax.axis_index("c") * NS + lax.axis_index("s")


def _deg_body(dst_r, degp, idxb, onesb, zb, acc):
  cid = lax.axis_index("c")
  tid = lax.axis_index("s")
  wid = _wid()

  _zero_rows(zb, G, DW)

  @pl.loop(0, G)
  def _(r):
    onesb[r, :] = jnp.ones((16,), jnp.float32)

  for k in range(RPT // G):
    pltpu.sync_copy(zb, acc.at[pl.ds(tid * RPT + k * G, G)])
  plsc.subcore_barrier()

  pltpu.sync_copy(dst_r.at[pl.ds(wid * GT, GT)], idxb)

  @pl.loop(0, GT)
  def _(j):
    pltpu.sync_copy(onesb, acc.at[idxb.at[j]], add=True)

  plsc.subcore_barrier()
  pltpu.sync_copy(acc.at[pl.ds(tid * RPT, RPT)],
                  degp.at[cid, pl.ds(tid * RPT, RPT)])


_deg_kernel = functools.partial(
    pl.kernel,
    out_type=jax.ShapeDtypeStruct((NC, NPAD, DW), jnp.float32),
    mesh=plsc.VectorSubcoreMesh(
        core_axis_name="c", subcore_axis_name="s", num_cores=NC,
        num_subcores=NS),
    compiler_params=pltpu.CompilerParams(use_tc_tiling_on_sc=False),
    scratch_types=[
        pltpu.VMEM((GT, G), jnp.int32),
        pltpu.VMEM((G, DW), jnp.float32),
        pltpu.VMEM((G, DW), jnp.float32),
        pltpu.VMEM_SHARED((NPAD, DW), jnp.float32),
    ],
)(_deg_body)


def _agg_body(hs, src_r, dst_r, p, sidx, didx, rows, zb, acc, width):
  cid = lax.axis_index("c")
  tid = lax.axis_index("s")
  wid = _wid()

  _zero_rows(zb, G, width)
  for k in range(RPT // G):
    pltpu.sync_copy(zb, acc.at[pl.ds(tid * RPT + k * G, G)])
  plsc.subcore_barrier()

  pltpu.sync_copy(src_r.at[pl.ds(wid * GT, GT)], sidx)
  pltpu.sync_copy(dst_r.at[pl.ds(wid * GT, GT)], didx)

  @pl.loop(0, GT)
  def _(j):
    pltpu.sync_copy(hs.at[sidx.at[j]], rows)
    pltpu.sync_copy(rows, acc.at[didx.at[j]], add=True)

  plsc.subcore_barrier()
  pltpu.sync_copy(acc.at[pl.ds(tid * RPT, RPT)],
                  p.at[cid, pl.ds(tid * RPT, RPT)])


def _make_agg(width):
  return functools.partial(
      pl.kernel,
      out_type=jax.ShapeDtypeStruct((NC, NPAD, width), jnp.float32),
      mesh=plsc.VectorSubcoreMesh(
          core_axis_name="c", subcore_axis_name="s", num_cores=NC,
          num_subcores=NS),
      compiler_params=pltpu.CompilerParams(use_tc_tiling_on_sc=False),
      scratch_types=[
          pltpu.VMEM((GT, G), jnp.int32),
          pltpu.VMEM((GT, G), jnp.int32),
          pltpu.VMEM((G, width), jnp.float32),
          pltpu.VMEM((G, width), jnp.float32),
          pltpu.VMEM_SHARED((NPAD, width), jnp.float32),
      ],
  )(functools.partial(_agg_body, width=width))


_agg64 = _make_agg(H)
_agg16 = _make_agg(C)


R = 1000


def _dinv_from(dp):
  deg = dp[0][:, 0:1] + dp[1][:, 0:1] + 1.0
  return lax.rsqrt(deg)


def _tc_a_body(x_ref, w1_ref, dp_ref, o_ref):
  dinv = _dinv_from(dp_ref)
  h = jnp.dot(x_ref[...], w1_ref[...], preferred_element_type=jnp.float32)
  o_ref[...] = h * dinv


def _tc_b_body(p_ref, h1s_ref, dp_ref, b1_ref, w2_ref, o_ref):
  dinv = _dinv_from(dp_ref)
  agg = p_ref[0] + p_ref[1] + h1s_ref[...]
  z = jnp.maximum(agg * dinv + b1_ref[...], 0.0)
  h2 = jnp.dot(z, w2_ref[...], preferred_element_type=jnp.float32)
  o_ref[...] = h2 * dinv


def _tc_c_body(q_ref, h2s_ref, dp_ref, b2_ref, o_ref):
  dinv = _dinv_from(dp_ref)
  logits = (q_ref[0] + q_ref[1] + h2s_ref[...]) * dinv + b2_ref[...]
  m = jnp.max(logits, axis=1, keepdims=True)
  e = jnp.exp(logits - m)
  s = jnp.sum(e, axis=1, keepdims=True)
  o_ref[...] = logits - m - jnp.log(s)


def _row_spec(width):
  return pl.BlockSpec((R, width), lambda i: (i, 0))


def _part_spec(width):
  return pl.BlockSpec((NC, R, width), lambda i: (0, i, 0))


def _full_spec(a, b):
  return pl.BlockSpec((a, b), lambda i: (0, 0))


_tc_a = pl.pallas_call(
    _tc_a_body,
    grid=(N // R,),
    in_specs=[_row_spec(D), _full_spec(D, H), _part_spec(DW)],
    out_specs=_row_spec(H),
    out_shape=jax.ShapeDtypeStruct((N, H), jnp.float32),
)

_tc_b = pl.pallas_call(
    _tc_b_body,
    grid=(N // R,),
    in_specs=[_part_spec(H), _row_spec(H), _part_spec(DW),
              _full_spec(1, H), _full_spec(H, C)],
    out_specs=_row_spec(C),
    out_shape=jax.ShapeDtypeStruct((N, C), jnp.float32),
)

_tc_c = pl.pallas_call(
    _tc_c_body,
    grid=(N // R,),
    in_specs=[_part_spec(C), _row_spec(C), _part_spec(DW), _full_spec(1, C)],
    out_specs=_row_spec(C),
    out_shape=jax.ShapeDtypeStruct((N, C), jnp.float32),
)


def kernel(x, edge_idx, W1, b1, W2, b2):
  pad = EPAD - E
  src = jnp.concatenate(
      [edge_idx[0], jnp.zeros((pad,), jnp.int32)]).reshape(NW * GT, G)
  dst = jnp.concatenate(
      [edge_idx[1], jnp.full((pad,), N, jnp.int32)]).reshape(NW * GT, G)

  degp = _deg_kernel(dst)
  h1s = _tc_a(x, W1, degp)
  p = _agg64(h1s, src, dst)
  h2s = _tc_b(p, h1s, degp, b1.reshape(1, H), W2)
  q = _agg16(h2s, src, dst)
  return _tc_c(q, h2s, degp, b2.reshape(1, C))
